# Initial kernel scaffold; baseline (speedup 1.0000x reference)
import jax
import jax.numpy as jnp
from jax.experimental import pallas as pl


def kernel(values, offsets, weights, tables):
    raise NotImplementedError("write your pallas kernel here")


# speedup vs baseline: 40.4645x; 40.4645x over previous
import functools

import jax
import jax.numpy as jnp
from jax import lax
from jax.experimental import pallas as pl
from jax.experimental.pallas import tpu as pltpu
from jax.experimental.pallas import tpu_sc as plsc

F = 26
B = 1024
V = 100000
D = 64
NC = 2
NS = 16
NW = NC * NS
RPW = B // NW
C = 128
L = 16


def _scal(vec, l):
    return jnp.squeeze(lax.slice_in_dim(vec, l, l + 1, axis=0))


def _body(values_h, offs0_h, offs1_h, weights_h, tables_h, out_h,
          offs_v, offs1_v, vals_v, w_v, idx_v, dst_v, rows_v, acc_sh,
          sem_g, sem_s):
    cid = lax.axis_index("c")
    sid = lax.axis_index("s")
    wid = cid * NS + sid
    i32 = jnp.int32
    mo = pl.multiple_of

    def zbody(i, c):
        for k in range(4):
            rows_v[i, pl.ds(k * L, L)] = jnp.zeros((L,), jnp.float32)
        return c
    lax.fori_loop(0, C, zbody, 0)
    abase = mo(sid * (RPW * F), RPW * F)
    for j in range(6):
        pltpu.sync_copy(rows_v, acc_sh.at[pl.ds(abase + j * C, C)])
    pltpu.sync_copy(rows_v.at[pl.ds(0, 64)], acc_sh.at[pl.ds(abase + 6 * C, 64)])

    pltpu.sync_copy(offs0_h, offs_v)
    pltpu.sync_copy(offs1_h, offs1_v)

    def fbody(f, carry):
        base = mo(f * B + wid * RPW, RPW)
        vec0 = offs_v[pl.ds(base, L)]
        vecA = offs1_v[pl.ds(base, L)]
        vecB = offs1_v[pl.ds(base + L, L)]
        start = _scal(vec0, 0)
        end = _scal(vecB, L - 1)
        start_al = mo(start - lax.rem(start, 8), 8)
        nch = lax.div(end - start_al + (C - 1), C)
        oj = ([_scal(vecA, l) for l in range(L)]
              + [_scal(vecB, l) for l in range(L)])
        fv = f * V

        def cbody(j, cc):
            p0 = mo(start_al + j * C, 8)
            pltpu.sync_copy(values_h.at[pl.ds(p0, C)], vals_v)
            pltpu.sync_copy(weights_h.at[pl.ds(p0, C)], w_v)

            def g1(g, c1):
                s = mo(g * L, L)
                pos = p0 + s + lax.iota(i32, L)
                v16 = vals_v[pl.ds(s, L)]
                idx_v[pl.ds(s, L)] = v16 + fv
                valid = (pos >= start) & (pos < end)
                w_v[pl.ds(s, L)] = jnp.where(valid, w_v[pl.ds(s, L)], 0.0)
                b16 = jnp.zeros((L,), i32)
                one = jnp.ones((L,), i32)
                zero = jnp.zeros((L,), i32)
                for o in oj:
                    b16 = b16 + jnp.where(pos >= o, one, zero)
                b16 = jnp.minimum(b16, RPW - 1)
                dst_v[pl.ds(s, L)] = abase + b16 * F + f
                return c1
            lax.fori_loop(0, C // L, g1, 0)

            pltpu.async_copy(tables_h.at[idx_v], rows_v, sem_g).wait()

            def g2(g, c2):
                s = mo(g * L, L)
                w16 = w_v[pl.ds(s, L)]
                for l in range(L):
                    wl = _scal(w16, l)
                    for k in range(4):
                        rows_v[s + l, pl.ds(k * L, L)] = (
                            rows_v[s + l, pl.ds(k * L, L)] * wl)
                return c2
            lax.fori_loop(0, C // L, g2, 0)

            pltpu.async_copy(rows_v, acc_sh.at[dst_v], sem_s, add=True).wait()
            return cc
        lax.fori_loop(0, nch, cbody, 0)
        return carry
    lax.fori_loop(0, F, fbody, 0)

    pltpu.sync_copy(acc_sh.at[pl.ds(abase, RPW * F)],
                    out_h.at[pl.ds(mo(wid * (RPW * F), RPW * F), RPW * F)])


@jax.jit
def kernel(values, offsets, weights, tables):
    n = values.shape[0]
    pad = C + 8
    values_p = jnp.concatenate([values, jnp.zeros((pad,), jnp.int32)])
    weights_p = jnp.concatenate([weights, jnp.zeros((pad,), jnp.float32)])
    opad = 48
    offs0 = jnp.concatenate([offsets, jnp.full((opad - 1,), n, jnp.int32)])
    offs1 = jnp.concatenate([offsets[1:], jnp.full((opad,), n, jnp.int32)])
    olen = int(offs0.shape[0])
    tflat = tables.reshape(F * V, D)

    mesh = plsc.VectorSubcoreMesh(core_axis_name="c", subcore_axis_name="s")
    f = pl.kernel(
        _body,
        out_type=jax.ShapeDtypeStruct((B * F, D), jnp.float32),
        mesh=mesh,
        compiler_params=pltpu.CompilerParams(use_tc_tiling_on_sc=False),
        scratch_types=[
            pltpu.VMEM((olen,), jnp.int32),
            pltpu.VMEM((olen,), jnp.int32),
            pltpu.VMEM((C,), jnp.int32),
            pltpu.VMEM((C,), jnp.float32),
            pltpu.VMEM((C,), jnp.int32),
            pltpu.VMEM((C,), jnp.int32),
            pltpu.VMEM((C, D), jnp.float32),
            pltpu.VMEM_SHARED((NS * RPW * F, D), jnp.float32),
            pltpu.SemaphoreType.DMA,
            pltpu.SemaphoreType.DMA,
        ],
    )
    out = f(values_p, offs0, offs1, weights_p, tflat)
    return out.reshape(B, F * D)
